# baseline (speedup 1.0000x reference)
import functools

import jax
import jax.numpy as jnp
from jax import lax
from jax.experimental import pallas as pl
from jax.experimental.pallas import tpu as pltpu
from jax.experimental.pallas import tpu_sc as plsc

_TBL = 1000
_BLK = 2048


@functools.lru_cache(maxsize=None)
def _make_sc_gather(B: int):
    info = plsc.get_sparse_core_info()
    NC, NS, L = info.num_cores, info.num_subcores, info.num_lanes
    NW = NC * NS
    b_per_w = B // NW
    assert B % (8 * NW) == 0

    mesh = plsc.VectorSubcoreMesh(core_axis_name="c", subcore_axis_name="s")

    @functools.partial(
        pl.kernel,
        mesh=mesh,
        out_type=[
            jax.ShapeDtypeStruct((B,), jnp.float32),
            jax.ShapeDtypeStruct((B,), jnp.float32),
        ],
        scratch_types=[
            pltpu.VMEM((b_per_w,), jnp.int32),
            pltpu.VMEM_SHARED((_TBL,), jnp.float32),
            pltpu.VMEM_SHARED((_TBL,), jnp.float32),
            pltpu.VMEM((b_per_w,), jnp.float32),
            pltpu.VMEM((b_per_w,), jnp.float32),
            pltpu.SemaphoreType.DMA,
            pltpu.SemaphoreType.DMA,
        ],
    )
    def sc_gather(t_hbm, tab1_hbm, tab2_hbm, s1_hbm, s2_hbm,
                  idx_v, tab1_sh, tab2_sh, s1_v, s2_v, sem1, sem2):
        sid = lax.axis_index("s")
        wid = sid * NC + lax.axis_index("c")
        base = wid * b_per_w
        pltpu.sync_copy(t_hbm.at[pl.ds(base, b_per_w)], idx_v)

        @pl.when(sid == 0)
        def _():
            pltpu.sync_copy(tab1_hbm, tab1_sh)
            pltpu.sync_copy(tab2_hbm, tab2_sh)

        plsc.subcore_barrier()
        cp1 = pltpu.async_copy(tab1_sh.at[idx_v], s1_v, sem1)
        cp2 = pltpu.async_copy(tab2_sh.at[idx_v], s2_v, sem2)
        cp1.wait()
        cp2.wait()
        pltpu.sync_copy(s1_v, s1_hbm.at[pl.ds(base, b_per_w)])
        pltpu.sync_copy(s2_v, s2_hbm.at[pl.ds(base, b_per_w)])

    return sc_gather


def _blend_body(s1_ref, s2_ref, xs_ref, xn_ref, o_ref):
    c1 = s1_ref[...].reshape(_BLK, 1)
    c2 = s2_ref[...].reshape(_BLK, 1)
    o_ref[...] = c1 * xs_ref[...] + c2 * xn_ref[...]


@functools.lru_cache(maxsize=None)
def _make_blend(B: int, D: int):
    nb = B // _BLK
    return pl.pallas_call(
        _blend_body,
        grid=(nb, 2),
        in_specs=[
            pl.BlockSpec((_BLK,), lambda i, j: (i,)),
            pl.BlockSpec((_BLK,), lambda i, j: (i,)),
            pl.BlockSpec((_BLK, D // 2), lambda i, j: (i, j)),
            pl.BlockSpec((_BLK, D // 2), lambda i, j: (i, j)),
        ],
        out_specs=pl.BlockSpec((_BLK, D // 2), lambda i, j: (i, j)),
        out_shape=jax.ShapeDtypeStruct((B, D), jnp.float32),
        compiler_params=pltpu.CompilerParams(
            dimension_semantics=("arbitrary", "arbitrary"),
        ),
    )


def kernel(x_start, x_noise, timesteps, sqrt_alphas_cumprod,
           sqrt_one_minus_alphas_cumprod):
    B, D = x_start.shape
    s1g, s2g = _make_sc_gather(B)(
        timesteps.astype(jnp.int32),
        sqrt_alphas_cumprod.astype(jnp.float32),
        sqrt_one_minus_alphas_cumprod.astype(jnp.float32),
    )
    return _make_blend(B, D)(s1g, s2g, x_start, x_noise)


# speedup vs baseline: 1.0377x; 1.0377x over previous
import functools

import jax
import jax.numpy as jnp
from jax import lax
from jax.experimental import pallas as pl
from jax.experimental.pallas import tpu as pltpu
from jax.experimental.pallas import tpu_sc as plsc

_TBL = 1000
_BLK = 2048


@functools.lru_cache(maxsize=None)
def _make_sc_gather(B: int):
    info = plsc.get_sparse_core_info()
    NC, NS, L = info.num_cores, info.num_subcores, info.num_lanes
    NW = NC * NS
    b_per_w = B // NW
    assert B % (8 * NW) == 0

    mesh = plsc.VectorSubcoreMesh(core_axis_name="c", subcore_axis_name="s")

    @functools.partial(
        pl.kernel,
        mesh=mesh,
        out_type=[
            jax.ShapeDtypeStruct((B,), jnp.float32),
            jax.ShapeDtypeStruct((B,), jnp.float32),
        ],
        scratch_types=[
            pltpu.VMEM((b_per_w,), jnp.int32),
            pltpu.VMEM_SHARED((_TBL,), jnp.float32),
            pltpu.VMEM_SHARED((_TBL,), jnp.float32),
            pltpu.VMEM((b_per_w,), jnp.float32),
            pltpu.VMEM((b_per_w,), jnp.float32),
            pltpu.SemaphoreType.DMA,
            pltpu.SemaphoreType.DMA,
        ],
    )
    def sc_gather(t_hbm, tab1_hbm, tab2_hbm, s1_hbm, s2_hbm,
                  idx_v, tab1_sh, tab2_sh, s1_v, s2_v, sem1, sem2):
        sid = lax.axis_index("s")
        wid = sid * NC + lax.axis_index("c")
        base = wid * b_per_w
        pltpu.sync_copy(t_hbm.at[pl.ds(base, b_per_w)], idx_v)

        @pl.when(sid == 0)
        def _():
            pltpu.sync_copy(tab1_hbm, tab1_sh)
            pltpu.sync_copy(tab2_hbm, tab2_sh)

        plsc.subcore_barrier()
        cp1 = pltpu.async_copy(tab1_sh.at[idx_v], s1_v, sem1)
        cp2 = pltpu.async_copy(tab2_sh.at[idx_v], s2_v, sem2)
        cp1.wait()
        cp2.wait()
        pltpu.sync_copy(s1_v, s1_hbm.at[pl.ds(base, b_per_w)])
        pltpu.sync_copy(s2_v, s2_hbm.at[pl.ds(base, b_per_w)])

    return sc_gather


def _blend_body(s1_ref, s2_ref, xs_ref, xn_ref, o_ref):
    c1 = s1_ref[...].reshape(_BLK, 1)
    c2 = s2_ref[...].reshape(_BLK, 1)
    o_ref[...] = c1 * xs_ref[...] + c2 * xn_ref[...]


@functools.lru_cache(maxsize=None)
def _make_blend(B: int, D: int):
    nb = B // _BLK
    return pl.pallas_call(
        _blend_body,
        grid=(nb,),
        in_specs=[
            pl.BlockSpec((_BLK,), lambda i: (i,)),
            pl.BlockSpec((_BLK,), lambda i: (i,)),
            pl.BlockSpec((_BLK, D), lambda i: (i, 0)),
            pl.BlockSpec((_BLK, D), lambda i: (i, 0)),
        ],
        out_specs=pl.BlockSpec((_BLK, D), lambda i: (i, 0)),
        out_shape=jax.ShapeDtypeStruct((B, D), jnp.float32),
        compiler_params=pltpu.CompilerParams(
            dimension_semantics=("arbitrary",),
        ),
    )


def kernel(x_start, x_noise, timesteps, sqrt_alphas_cumprod,
           sqrt_one_minus_alphas_cumprod):
    B, D = x_start.shape
    s1g, s2g = _make_sc_gather(B)(
        timesteps.astype(jnp.int32),
        sqrt_alphas_cumprod.astype(jnp.float32),
        sqrt_one_minus_alphas_cumprod.astype(jnp.float32),
    )
    return _make_blend(B, D)(s1g, s2g, x_start, x_noise)


# speedup vs baseline: 1.0466x; 1.0085x over previous
import functools

import jax
import jax.numpy as jnp
from jax import lax
from jax.experimental import pallas as pl
from jax.experimental.pallas import tpu as pltpu
from jax.experimental.pallas import tpu_sc as plsc

_TBL = 1000
_BLK = 2048


@functools.lru_cache(maxsize=None)
def _make_sc_gather(B: int):
    info = plsc.get_sparse_core_info()
    NC, NS, L = info.num_cores, info.num_subcores, info.num_lanes
    NW = NC * NS
    b_per_w = B // NW
    assert B % (8 * NW) == 0

    mesh = plsc.VectorSubcoreMesh(core_axis_name="c", subcore_axis_name="s")

    @functools.partial(
        pl.kernel,
        mesh=mesh,
        out_type=[
            jax.ShapeDtypeStruct((B,), jnp.float32),
            jax.ShapeDtypeStruct((B,), jnp.float32),
        ],
        scratch_types=[
            pltpu.VMEM((b_per_w,), jnp.int32),
            pltpu.VMEM_SHARED((_TBL,), jnp.float32),
            pltpu.VMEM_SHARED((_TBL,), jnp.float32),
            pltpu.VMEM((b_per_w,), jnp.float32),
            pltpu.VMEM((b_per_w,), jnp.float32),
            pltpu.SemaphoreType.DMA,
            pltpu.SemaphoreType.DMA,
            pltpu.SemaphoreType.DMA,
        ],
    )
    def sc_gather(t_hbm, tab1_hbm, tab2_hbm, s1_hbm, s2_hbm,
                  idx_v, tab1_sh, tab2_sh, s1_v, s2_v, sem1, sem2, sem3):
        sid = lax.axis_index("s")
        wid = sid * NC + lax.axis_index("c")
        base = wid * b_per_w
        idx_cp = pltpu.async_copy(t_hbm.at[pl.ds(base, b_per_w)], idx_v, sem3)

        @pl.when(sid == 0)
        def _():
            pltpu.sync_copy(tab1_hbm, tab1_sh)
            pltpu.sync_copy(tab2_hbm, tab2_sh)

        plsc.subcore_barrier()
        idx_cp.wait()
        cp1 = pltpu.async_copy(tab1_sh.at[idx_v], s1_v, sem1)
        cp2 = pltpu.async_copy(tab2_sh.at[idx_v], s2_v, sem2)
        cp1.wait()
        cp2.wait()
        out1 = pltpu.async_copy(s1_v, s1_hbm.at[pl.ds(base, b_per_w)], sem1)
        out2 = pltpu.async_copy(s2_v, s2_hbm.at[pl.ds(base, b_per_w)], sem2)
        out1.wait()
        out2.wait()

    return sc_gather


def _blend_body(s1_ref, s2_ref, xs_ref, xn_ref, o_ref):
    c1 = s1_ref[...].reshape(_BLK, 1)
    c2 = s2_ref[...].reshape(_BLK, 1)
    o_ref[...] = c1 * xs_ref[...] + c2 * xn_ref[...]


@functools.lru_cache(maxsize=None)
def _make_blend(B: int, D: int):
    nb = B // _BLK
    return pl.pallas_call(
        _blend_body,
        grid=(nb,),
        in_specs=[
            pl.BlockSpec((_BLK,), lambda i: (i,)),
            pl.BlockSpec((_BLK,), lambda i: (i,)),
            pl.BlockSpec((_BLK, D), lambda i: (i, 0)),
            pl.BlockSpec((_BLK, D), lambda i: (i, 0)),
        ],
        out_specs=pl.BlockSpec((_BLK, D), lambda i: (i, 0)),
        out_shape=jax.ShapeDtypeStruct((B, D), jnp.float32),
        compiler_params=pltpu.CompilerParams(
            dimension_semantics=("arbitrary",),
        ),
    )


def kernel(x_start, x_noise, timesteps, sqrt_alphas_cumprod,
           sqrt_one_minus_alphas_cumprod):
    B, D = x_start.shape
    s1g, s2g = _make_sc_gather(B)(
        timesteps.astype(jnp.int32),
        sqrt_alphas_cumprod.astype(jnp.float32),
        sqrt_one_minus_alphas_cumprod.astype(jnp.float32),
    )
    return _make_blend(B, D)(s1g, s2g, x_start, x_noise)
